# baseline (speedup 1.0000x reference)
import jax
import jax.numpy as jnp
from jax import lax
from jax.experimental import pallas as pl

_VOCAB = 1000
_ROWS_PER_BLK = 512


def _onehot_body(x_ref, o_ref):
    idx = x_ref[0, 0, :]
    col = lax.broadcasted_iota(jnp.int32, (_ROWS_PER_BLK, _VOCAB), 1)
    o_ref[...] = (col == idx[:, None]).astype(jnp.float32)


def kernel(x, W):
    b, h = x.shape
    n = b * h
    nblk = n // _ROWS_PER_BLK
    x3 = x.reshape(nblk, 1, _ROWS_PER_BLK)
    out = pl.pallas_call(
        _onehot_body,
        grid=(nblk,),
        in_specs=[pl.BlockSpec((1, 1, _ROWS_PER_BLK), lambda i: (i, 0, 0))],
        out_specs=pl.BlockSpec((_ROWS_PER_BLK, _VOCAB), lambda i: (i, 0)),
        out_shape=jax.ShapeDtypeStruct((n, _VOCAB), jnp.float32),
    )(x3)
    return out.reshape(b, h, _VOCAB)


# speedup vs baseline: 1.6609x; 1.6609x over previous
import jax
import jax.numpy as jnp
from jax import lax
from jax.experimental import pallas as pl

_VOCAB = 1000
_B_BLK = 128


def _onehot_body(x_ref, o_ref):
    h = x_ref.shape[1]
    idx = x_ref[...]
    col = lax.broadcasted_iota(jnp.int32, (_B_BLK, h, _VOCAB), 2)
    o_ref[...] = (col == idx[:, :, None]).astype(jnp.float32)


def kernel(x, W):
    b, h = x.shape
    nblk = b // _B_BLK
    return pl.pallas_call(
        _onehot_body,
        grid=(nblk,),
        in_specs=[pl.BlockSpec((_B_BLK, h), lambda i: (i, 0))],
        out_specs=pl.BlockSpec((_B_BLK, h, _VOCAB), lambda i: (i, 0, 0)),
        out_shape=jax.ShapeDtypeStruct((b, h, _VOCAB), jnp.float32),
    )(x)
